# baseline (speedup 1.0000x reference)
import functools

import jax
import jax.numpy as jnp
from jax import lax
from jax.experimental import pallas as pl
from jax.experimental.pallas import tpu as pltpu
from jax.experimental.pallas import tpu_sc as plsc

EMB = 128
LANES = 16

NC, NS = 2, 16
NW = NC * NS
CH = 128
NBUF = 6


def _make_sc_embed(batch, seq):
    rows = batch * seq
    rpw = rows // NW
    nch = rpw // CH
    mesh = plsc.VectorSubcoreMesh(core_axis_name="c", subcore_axis_name="s")

    @functools.partial(
        pl.kernel,
        out_type=jax.ShapeDtypeStruct((batch, seq, EMB), jnp.float32),
        mesh=mesh,
        scratch_types=[
            pltpu.VMEM((rpw,), jnp.int32),
            pltpu.VMEM((rpw,), jnp.int32),
            pltpu.VMEM((2, EMB), jnp.float32),
            pltpu.VMEM((NBUF, CH, EMB), jnp.float32),
            pltpu.SemaphoreType.DMA((NBUF,)),
            pltpu.SemaphoreType.DMA((NBUF,)),
            pltpu.SemaphoreType.DMA((NBUF,)),
        ],
    )
    def sc_embed(tok_hbm, typ_hbm, word_hbm, pos_hbm, type_hbm, out_hbm,
                 tok_v, typ_v, ttab, buf, sem_p, sem_w, sem_s):
        wid = lax.axis_index("s") * NC + lax.axis_index("c")
        b = wid // (seq // rpw)
        soff = lax.rem(wid, seq // rpw) * rpw

        pltpu.sync_copy(tok_hbm.at[b, pl.ds(soff, rpw)], tok_v)
        pltpu.sync_copy(typ_hbm.at[b, pl.ds(soff, rpw)], typ_v)
        pltpu.sync_copy(type_hbm, ttab)

        poss, words, stores = {}, {}, {}

        def start_pos(j):
            slot = j % NBUF
            poss[j] = pltpu.async_copy(
                pos_hbm.at[pl.ds(soff + j * CH, CH)],
                buf.at[slot], sem_p.at[slot])

        def finish_chunk(j):
            slot = j % NBUF
            words.pop(j).wait()

            def tgroup(g, carry):
                row0 = g * LANES
                tv = typ_v[pl.ds(j * CH + row0, LANES)]
                for k in range(LANES):
                    t = tv[k]
                    for v in range(EMB // LANES):
                        sl = pl.ds(v * LANES, LANES)
                        plsc.addupdate(buf.at[slot, row0 + k, sl], ttab[t, sl])
                return carry

            lax.fori_loop(0, CH // LANES, tgroup, 0)
            stores[j] = pltpu.async_copy(
                buf.at[slot], out_hbm.at[b, pl.ds(soff + j * CH, CH)],
                sem_s.at[slot])

        for j in range(min(NBUF, nch)):
            start_pos(j)

        for j in range(nch):
            slot = j % NBUF
            poss.pop(j).wait()
            words[j] = pltpu.async_copy(
                word_hbm.at[tok_v.at[pl.ds(j * CH, CH)]],
                buf.at[slot], sem_w.at[slot], add=True)
            if j >= 2:
                finish_chunk(j - 2)
            if j >= 3 and j - 3 + NBUF < nch:
                stores.pop(j - 3).wait()
                start_pos(j - 3 + NBUF)

        for j in sorted(words):
            finish_chunk(j)
        for j in sorted(stores):
            stores.pop(j).wait()

    return sc_embed


def kernel(input_tokens, input_token_types, word_table, pos_table, type_table):
    batch, seq = input_tokens.shape
    return _make_sc_embed(batch, seq)(
        input_tokens, input_token_types, word_table, pos_table, type_table)


# speedup vs baseline: 2.0161x; 2.0161x over previous
import functools

import jax
import jax.numpy as jnp
from jax import lax
from jax.experimental import pallas as pl
from jax.experimental.pallas import tpu as pltpu
from jax.experimental.pallas import tpu_sc as plsc

SEQ = 2048
EMB = 128
NTYP = 2
LANES = 16

NC, NS = 2, 16
NW = NC * NS
CH = 128
NBUF = 6


def _comb_body(pos_ref, type_ref, out_ref):
    out_ref[...] = pos_ref[...][None, :, :] + type_ref[...][:, None, :]


def _build_comb(pos_table, type_table):
    comb = pl.pallas_call(
        _comb_body,
        out_shape=jax.ShapeDtypeStruct((NTYP, SEQ, EMB), jnp.float32),
    )(pos_table, type_table)
    return comb.reshape(NTYP * SEQ, EMB)


def _make_sc_embed(batch, seq):
    rows = batch * seq
    rpw = rows // NW
    nch = rpw // CH
    mesh = plsc.VectorSubcoreMesh(core_axis_name="c", subcore_axis_name="s")

    @functools.partial(
        pl.kernel,
        out_type=jax.ShapeDtypeStruct((batch, seq, EMB), jnp.float32),
        mesh=mesh,
        scratch_types=[
            pltpu.VMEM((rpw,), jnp.int32),
            pltpu.VMEM((rpw,), jnp.int32),
            pltpu.VMEM((NBUF, CH, EMB), jnp.float32),
            pltpu.VMEM_SHARED((NTYP * (SEQ // NC), EMB), jnp.float32),
            pltpu.SemaphoreType.DMA,
            pltpu.SemaphoreType.DMA((NBUF,)),
            pltpu.SemaphoreType.DMA((NBUF,)),
            pltpu.SemaphoreType.DMA((NBUF,)),
        ],
    )
    def sc_embed(tok_hbm, typ_hbm, word_hbm, comb_hbm, out_hbm,
                 tok_v, cidx_v, buf, shared, sem_g, sem_c, sem_w, sem_s):
        wid = lax.axis_index("s") * NC + lax.axis_index("c")
        sid = lax.axis_index("s")
        b = wid // (seq // rpw)
        soff = pl.multiple_of(
            lax.rem(wid, seq // rpw) * rpw, CH)

        core = lax.axis_index("c")
        stage = [
            pltpu.make_async_copy(
                comb_hbm.at[t * NC + core],
                shared.at[pl.ds(t * rpw, rpw)], sem_g)
            for t in range(NTYP)
        ]

        @pl.when(sid == 0)
        def _():
            for d in stage:
                d.start()

        pltpu.sync_copy(tok_hbm.at[b, pl.ds(soff, rpw)], tok_v)
        pltpu.sync_copy(typ_hbm.at[b, pl.ds(soff, rpw)], cidx_v)

        for v in range(rpw // LANES):
            sl = pl.ds(v * LANES, LANES)
            s_vec = lax.iota(jnp.int32, LANES) + (v * LANES)
            cidx_v[sl] = cidx_v[sl] * rpw + s_vec

        combs, words, stores = {}, {}, {}

        def start_word(j):
            slot = j % NBUF
            words[j] = pltpu.async_copy(
                word_hbm.at[tok_v.at[pl.ds(j * CH, CH)]],
                buf.at[slot], sem_w.at[slot])

        def start_store(j):
            slot = j % NBUF
            combs.pop(j).wait()
            stores[j] = pltpu.async_copy(
                buf.at[slot], out_hbm.at[b, pl.ds(soff + j * CH, CH)],
                sem_s.at[slot])

        for j in range(min(NBUF, nch)):
            start_word(j)

        @pl.when(sid == 0)
        def _():
            for d in stage:
                d.wait()

        plsc.subcore_barrier()

        for j in range(nch):
            slot = j % NBUF
            words.pop(j).wait()
            combs[j] = pltpu.async_copy(
                shared.at[cidx_v.at[pl.ds(j * CH, CH)]],
                buf.at[slot], sem_c.at[slot], add=True)
            if j >= 2:
                start_store(j - 2)
            if j >= 3 and j - 3 + NBUF < nch:
                stores.pop(j - 3).wait()
                start_word(j - 3 + NBUF)

        for j in sorted(combs):
            start_store(j)
        for j in sorted(stores):
            stores.pop(j).wait()

    return sc_embed


def kernel(input_tokens, input_token_types, word_table, pos_table, type_table):
    batch, seq = input_tokens.shape
    comb = _build_comb(pos_table, type_table)
    rpw = batch * seq // NW
    comb = comb.reshape(NTYP * NC, rpw, EMB)
    return _make_sc_embed(batch, seq)(
        input_tokens, input_token_types, word_table, comb)
